# baseline (speedup 1.0000x reference)
import functools

import jax
import jax.numpy as jnp
from jax import lax
from jax.experimental import pallas as pl
from jax.experimental.pallas import tpu as pltpu
from jax.experimental.pallas import tpu_sc as plsc

B, S, F = 2, 8192, 2048
L = 16
NC, NS = 2, 16
NW = NC * NS
GPW = 8
FW = GPW * L
WPB = NW // B
CHUNK = 256
NCHUNK = S // CHUNK

_mesh = plsc.VectorSubcoreMesh(core_axis_name="c", subcore_axis_name="s")


@functools.partial(
    pl.kernel,
    mesh=_mesh,
    out_type=jax.ShapeDtypeStruct((B, S, F), jnp.float32),
    compiler_params=pltpu.CompilerParams(
        use_tc_tiling_on_sc=False, needs_layout_passes=False),
    scratch_types=[pltpu.VMEM((CHUNK, FW), jnp.float32)],
)
def _cumsum_sc(x_hbm, out_hbm, buf):
    wid = lax.axis_index("s") * NC + lax.axis_index("c")
    b = wid // WPB
    f0 = (wid % WPB) * FW

    def chunk_body(ci, accs):
        row0 = ci * CHUNK
        pltpu.sync_copy(
            x_hbm.at[b, pl.ds(row0, CHUNK), pl.ds(f0, FW)], buf)

        def row_body(i, accs):
            new = []
            for g in range(GPW):
                v = buf[i, pl.ds(g * L, L)]
                buf[i, pl.ds(g * L, L)] = accs[g]
                new.append(accs[g] + v)
            return tuple(new)

        accs = lax.fori_loop(0, CHUNK, row_body, accs)
        pltpu.sync_copy(
            buf, out_hbm.at[b, pl.ds(row0, CHUNK), pl.ds(f0, FW)])
        return accs

    zeros = tuple(jnp.zeros((L,), jnp.float32) for _ in range(GPW))
    lax.fori_loop(0, NCHUNK, chunk_body, zeros)


def kernel(x):
    return _cumsum_sc(x)


# speedup vs baseline: 1.1560x; 1.1560x over previous
import functools

import jax
import jax.numpy as jnp
from jax import lax
from jax.experimental import pallas as pl
from jax.experimental.pallas import tpu as pltpu
from jax.experimental.pallas import tpu_sc as plsc

B, S, F = 2, 8192, 2048
L = 16
NC, NS = 2, 16
NW = NC * NS
GPW = 8
FW = GPW * L
WPB = NW // B
CHUNK = 256
NCHUNK = S // CHUNK
NBUF = 3

_mesh = plsc.VectorSubcoreMesh(core_axis_name="c", subcore_axis_name="s")


@functools.partial(
    pl.kernel,
    mesh=_mesh,
    out_type=jax.ShapeDtypeStruct((B, S, F), jnp.float32),
    compiler_params=pltpu.CompilerParams(
        use_tc_tiling_on_sc=False, needs_layout_passes=False),
    scratch_types=(
        [pltpu.VMEM((NBUF, CHUNK, FW), jnp.float32)]
        + [pltpu.SemaphoreType.DMA] * (2 * NBUF)
    ),
)
def _cumsum_sc(x_hbm, out_hbm, buf, *sems):
    in_sems, out_sems = sems[:NBUF], sems[NBUF:]
    wid = lax.axis_index("s") * NC + lax.axis_index("c")
    b = wid // WPB
    f0 = (wid % WPB) * FW

    def src(ci):
        return x_hbm.at[b, pl.ds(ci * CHUNK, CHUNK), pl.ds(f0, FW)]

    def dst(ci):
        return out_hbm.at[b, pl.ds(ci * CHUNK, CHUNK), pl.ds(f0, FW)]

    for ci in range(NBUF - 1):
        pltpu.async_copy(src(ci), buf.at[ci % NBUF], in_sems[ci % NBUF])

    accs = tuple(jnp.zeros((L,), jnp.float32) for _ in range(GPW))
    for ci in range(NCHUNK):
        k = ci % NBUF
        pltpu.make_async_copy(src(ci), buf.at[k], in_sems[k]).wait()

        def row_body(i, accs, k=k):
            new = []
            for g in range(GPW):
                v = buf[k, i, pl.ds(g * L, L)]
                buf[k, i, pl.ds(g * L, L)] = accs[g]
                new.append(accs[g] + v)
            return tuple(new)

        accs = lax.fori_loop(0, CHUNK, row_body, accs)
        pltpu.async_copy(buf.at[k], dst(ci), out_sems[k])
        nci = ci + (NBUF - 1)
        if nci < NCHUNK:
            nk = nci % NBUF
            if nci - NBUF >= 0:
                pltpu.make_async_copy(
                    buf.at[nk], dst(nci - NBUF), out_sems[nk]).wait()
            pltpu.async_copy(src(nci), buf.at[nk], in_sems[nk])

    for ci in range(NCHUNK - NBUF, NCHUNK):
        k = ci % NBUF
        pltpu.make_async_copy(buf.at[k], dst(ci), out_sems[k]).wait()


def kernel(x):
    return _cumsum_sc(x)
